# Initial kernel scaffold; baseline (speedup 1.0000x reference)
import jax
import jax.numpy as jnp
from jax.experimental import pallas as pl


def kernel(x, edge_index, batch_index, W0, b0, W1, b1, W2, b2, W3, b3, Wout, bout):
    raise NotImplementedError("write your pallas kernel here")


# speedup vs baseline: 16.1613x; 16.1613x over previous
import functools

import jax
import jax.numpy as jnp
from jax import lax
from jax.experimental import pallas as pl
from jax.experimental.pallas import tpu as pltpu
from jax.experimental.pallas import tpu_sc as plsc

N = 10000
E = 640000
H = 128
G = 256

NC = 2
NS = 16
NW = NC * NS
CH = 100
EPW = E // NW
CPW = EPW // CH
GRP = 8
ZR = 624
TAIL = N - NS * ZR


@functools.cache
def _sc_mesh():
    return plsc.VectorSubcoreMesh(
        core_axis_name="c", subcore_axis_name="s",
        num_cores=NC, num_subcores=NS)


@functools.cache
def _sc_degree_kernel():
    return pl.kernel(
        _sc_degree_body,
        out_type=jax.ShapeDtypeStruct((NW, N), jnp.float32),
        mesh=_sc_mesh(),
        scratch_types=[
            pltpu.VMEM((EPW,), jnp.int32),
            pltpu.VMEM((N,), jnp.float32),
        ],
        compiler_params=pltpu.CompilerParams(needs_layout_passes=False),
    )


def _sc_degree_body(dst1, deg_out, dst_v, cnt_v):
    c = lax.axis_index("c")
    s = lax.axis_index("s")
    w = c * NS + s
    z16 = jnp.zeros((16,), jnp.float32)

    def zero(k, _):
        cnt_v[pl.ds(k * 16, 16)] = z16
        return ()

    lax.fori_loop(0, N // 16, zero, ())
    pltpu.sync_copy(dst1.at[pl.ds(w * EPW, EPW)], dst_v)
    o16 = jnp.ones((16,), jnp.float32)

    def body(k, _):
        idx = dst_v[pl.ds(k * 16, 16)]
        plsc.addupdate_scatter(cnt_v, [idx], o16)
        return ()

    lax.fori_loop(0, EPW // 16, body, ())
    pltpu.sync_copy(cnt_v, deg_out.at[w])


@functools.cache
def _sc_scatter_kernel():
    return pl.kernel(
        _sc_scatter_body,
        out_type=jax.ShapeDtypeStruct((NC, N, H), jnp.float32),
        mesh=_sc_mesh(),
        scratch_types=[
            pltpu.VMEM((GRP, CH), jnp.int32),
            pltpu.VMEM((GRP, CH), jnp.int32),
            pltpu.VMEM((CH, H), jnp.float32),
            pltpu.VMEM_SHARED((N, H), jnp.float32),
        ],
    )


def _sc_scatter_body(hws, src2, dst2, zrows, agg_out, src_v, dst_v, rows_v, acc):
    c = lax.axis_index("c")
    s = lax.axis_index("s")
    w = c * NS + s
    pltpu.sync_copy(zrows, acc.at[pl.ds(s * ZR, ZR)])

    @pl.when(s == 0)
    def _():
        pltpu.sync_copy(zrows.at[pl.ds(0, TAIL)], acc.at[pl.ds(NS * ZR, TAIL)])

    plsc.subcore_barrier()

    def group(g, _):
        base = w * CPW + g * GRP
        pltpu.sync_copy(src2.at[pl.ds(base, GRP)], src_v)
        pltpu.sync_copy(dst2.at[pl.ds(base, GRP)], dst_v)

        def body(i, _):
            pltpu.sync_copy(hws.at[src_v.at[i]], rows_v)
            pltpu.sync_copy(rows_v, acc.at[dst_v.at[i]], add=True)
            return ()

        lax.fori_loop(0, GRP, body, ())
        return ()

    lax.fori_loop(0, CPW // GRP, group, ())
    plsc.subcore_barrier()
    pltpu.sync_copy(acc.at[pl.ds(s * ZR, ZR)],
                    agg_out.at[c, pl.ds(s * ZR, ZR)])

    @pl.when(s == 0)
    def _():
        pltpu.sync_copy(acc.at[pl.ds(NS * ZR, TAIL)],
                        agg_out.at[c, pl.ds(NS * ZR, TAIL)])


_RB = 1000


def _tc_pre_body(deg_ref, x_ref, w_ref, dinv_ref, hws_ref):
    deg = jnp.sum(deg_ref[...], axis=1) + 1.0
    dinv = lax.rsqrt(deg)[:, None]
    dinv_ref[...] = jnp.broadcast_to(dinv, dinv_ref.shape)
    hw = jnp.dot(x_ref[...], w_ref[...], preferred_element_type=jnp.float32)
    hws_ref[...] = hw * dinv


def _tc_layer_body(agg_ref, hws_ref, dinv_ref, b_ref, w_ref, out_ref):
    dinv = dinv_ref[...][:, 0:1]
    a = (agg_ref[0] + agg_ref[1] + hws_ref[...]) * dinv + b_ref[...]
    h = jnp.tanh(a)
    out_ref[...] = jnp.dot(
        h, w_ref[...], preferred_element_type=jnp.float32) * dinv


def _tc_last_body(agg_ref, hws_ref, dinv_ref, b_ref, out_ref):
    dinv = dinv_ref[...][:, 0:1]
    a = (agg_ref[0] + agg_ref[1] + hws_ref[...]) * dinv + b_ref[...]
    out_ref[...] = jnp.tanh(a)


def _row_specs():
    agg = pl.BlockSpec((NC, _RB, H), lambda i: (0, i, 0))
    hws = pl.BlockSpec((_RB, H), lambda i: (i, 0))
    dinv = pl.BlockSpec((_RB, 16), lambda i: (i, 0))
    b = pl.BlockSpec((1, H), lambda i: (0, 0))
    w = pl.BlockSpec((H, H), lambda i: (0, 0))
    return agg, hws, dinv, b, w


def _tc_pre(deg32, x, W0):
    agg, hws, dinv, _, w = _row_specs()
    deg = pl.BlockSpec((_RB, NW), lambda i: (i, 0))
    return pl.pallas_call(
        _tc_pre_body,
        grid=(N // _RB,),
        in_specs=[deg, hws, w],
        out_specs=[dinv, hws],
        out_shape=[jax.ShapeDtypeStruct((N, 16), jnp.float32),
                   jax.ShapeDtypeStruct((N, H), jnp.float32)],
    )(deg32, x, W0)


def _tc_layer(agg2, hws_in, dinv16, b, Wn):
    agg, hws, dinv, bsp, w = _row_specs()
    return pl.pallas_call(
        _tc_layer_body,
        grid=(N // _RB,),
        in_specs=[agg, hws, dinv, bsp, w],
        out_specs=hws,
        out_shape=jax.ShapeDtypeStruct((N, H), jnp.float32),
    )(agg2, hws_in, dinv16, b, Wn)


def _tc_last(agg2, hws_in, dinv16, b):
    agg, hws, dinv, bsp, _ = _row_specs()
    return pl.pallas_call(
        _tc_last_body,
        grid=(N // _RB,),
        in_specs=[agg, hws, dinv, bsp],
        out_specs=hws,
        out_shape=jax.ShapeDtypeStruct((N, H), jnp.float32),
    )(agg2, hws_in, dinv16, b)


def _tc_pool_body(h_ref, batch_ref, wout_ref, bout_ref, out_ref, hid_ref,
                  mx_ref, sm_ref, cn_ref):
    mx_ref[...] = jnp.full((G, H), -3.4e38, jnp.float32)
    sm_ref[...] = jnp.zeros((G, H), jnp.float32)
    cn_ref[...] = jnp.zeros((G, H), jnp.float32)

    def body(i, _):
        g = batch_ref[i]
        row = h_ref[pl.ds(i, 1), :]
        mx_ref[pl.ds(g, 1), :] = jnp.maximum(mx_ref[pl.ds(g, 1), :], row)
        sm_ref[pl.ds(g, 1), :] = sm_ref[pl.ds(g, 1), :] + row
        cn_ref[pl.ds(g, 1), :] = cn_ref[pl.ds(g, 1), :] + 1.0
        return ()

    lax.fori_loop(0, N, body, ())
    cnt = cn_ref[...][:, 0:1]
    mx = jnp.where(cnt > 0, mx_ref[...], 0.0)
    mean = sm_ref[...] / jnp.maximum(cnt, 1.0)
    hidden = jnp.concatenate([mx, mean], axis=1)
    hid_ref[...] = hidden
    out_ref[...] = jnp.dot(
        hidden, wout_ref[...], preferred_element_type=jnp.float32) + bout_ref[...]


def _tc_pool(h4, batch_index, Wout, bout):
    return pl.pallas_call(
        _tc_pool_body,
        in_specs=[pl.BlockSpec(memory_space=pltpu.VMEM),
                  pl.BlockSpec(memory_space=pltpu.SMEM),
                  pl.BlockSpec(memory_space=pltpu.VMEM),
                  pl.BlockSpec(memory_space=pltpu.VMEM)],
        out_specs=[pl.BlockSpec(memory_space=pltpu.VMEM),
                   pl.BlockSpec(memory_space=pltpu.VMEM)],
        out_shape=[jax.ShapeDtypeStruct((G, 1), jnp.float32),
                   jax.ShapeDtypeStruct((G, 2 * H), jnp.float32)],
        scratch_shapes=[pltpu.VMEM((G, H), jnp.float32),
                        pltpu.VMEM((G, H), jnp.float32),
                        pltpu.VMEM((G, H), jnp.float32)],
    )(h4, batch_index, Wout, bout)


def kernel(x, edge_index, batch_index, W0, b0, W1, b1, W2, b2, W3, b3,
           Wout, bout):
    src2 = edge_index[0].reshape(E // CH, CH)
    dst2 = edge_index[1].reshape(E // CH, CH)
    zrows = jnp.zeros((ZR, H), jnp.float32)

    deg32 = _sc_degree_kernel()(edge_index[1])
    dinv16, hws = _tc_pre(deg32.T, x, W0)

    for b, Wn in ((b0, W1), (b1, W2), (b2, W3)):
        agg2 = _sc_scatter_kernel()(hws, src2, dst2, zrows)
        hws = _tc_layer(agg2, hws, dinv16, b.reshape(1, H), Wn)

    agg2 = _sc_scatter_kernel()(hws, src2, dst2, zrows)
    h4 = _tc_last(agg2, hws, dinv16, b3.reshape(1, H))

    out, hidden = _tc_pool(h4, batch_index, Wout, bout)
    return out, hidden
